# Initial kernel scaffold; baseline (speedup 1.0000x reference)
import jax
import jax.numpy as jnp
from jax.experimental import pallas as pl


def kernel(x, edge_index, batch, label, W1, W2):
    raise NotImplementedError("write your pallas kernel here")


# speedup vs baseline: 69.1403x; 69.1403x over previous
import functools

import jax
import jax.numpy as jnp
from jax import lax
from jax.experimental import pallas as pl
from jax.experimental.pallas import tpu as pltpu
from jax.experimental.pallas import tpu_sc as plsc

N = 10000
D = 128
H = 128
E = 320000
G = 64

NROW = 640
NPAD = NROW * 16
CHUNK = 125
NCHUNKS = E // CHUNK
NC, NS = 2, 16
NW = NC * NS
CPW = NCHUNKS // NW


def _reduce_partials(sid, cid, buf, partials_sh, stripebuf, outbuf,
                     out0_hbm, out1_hbm):
    pltpu.sync_copy(buf, partials_sh.at[sid])
    plsc.subcore_barrier()
    base = sid * (NPAD // NS)
    for t in range(NS):
        pltpu.sync_copy(
            partials_sh.at[t, pl.ds(base, NPAD // NS)], stripebuf.at[t]
        )

    def _sum(v, c):
        tot = stripebuf[0, pl.ds(v * 16, 16)]
        for t in range(1, NS):
            tot = tot + stripebuf[t, pl.ds(v * 16, 16)]
        outbuf[pl.ds(v * 16, 16)] = tot
        return c

    lax.fori_loop(0, NPAD // NS // 16, _sum, 0, unroll=4)

    @pl.when(cid == 0)
    def _():
        pltpu.sync_copy(outbuf, out0_hbm.at[pl.ds(base, NPAD // NS)])

    @pl.when(cid == 1)
    def _():
        pltpu.sync_copy(outbuf, out1_hbm.at[pl.ds(base, NPAD // NS)])


EPW = E // NW


def _sc_deg_body(ei_hbm, deg0_out, deg1_out, dstv, degbuf, stripebuf, outbuf,
                 partials_sh):
    cid = lax.axis_index("c")
    sid = lax.axis_index("s")
    wid = cid * NS + sid

    def _zero(j, c):
        degbuf[pl.ds(j * 16, 16)] = jnp.zeros((16,), jnp.float32)
        return c

    lax.fori_loop(0, NPAD // 16, _zero, 0, unroll=8)

    pltpu.sync_copy(ei_hbm.at[pl.ds(E + wid * EPW, EPW)], dstv)
    ones16 = jnp.ones((16,), jnp.float32)

    def _step(j, c):
        dv = dstv[pl.ds(j * 16, 16)]
        plsc.addupdate_scatter(degbuf, [dv], ones16)
        return c

    lax.fori_loop(0, EPW // 16, _step, 0, unroll=8)
    _reduce_partials(sid, cid, degbuf, partials_sh, stripebuf, outbuf,
                     deg0_out, deg1_out)


_NB = 4


def _sc_scatter_body(y1_hbm, ei3_hbm, out0_hbm, out1_hbm,
                     srcbuf, dstbuf, rows0, rows1, rows2, rows3, zbuf, acc_sh,
                     sem0, sem1, sem2, sem3):
    cid = lax.axis_index("c")
    sid = lax.axis_index("s")
    wid = cid * NS + sid

    def _zr(j, c):
        for jj in range(4):
            zbuf[j, pl.ds(jj * 32, 32)] = jnp.zeros((32,), jnp.bfloat16)
        return c

    lax.fori_loop(0, 40, _zr, 0)

    def _zs(k, c):
        pltpu.sync_copy(zbuf, acc_sh.at[pl.ds(sid * (NPAD // NS) + k * 40, 40)])
        return c

    lax.fori_loop(0, NPAD // NS // 40, _zs, 0)

    base = wid * CPW
    pltpu.sync_copy(ei3_hbm.at[0, pl.ds(base, CPW)], srcbuf)
    pltpu.sync_copy(ei3_hbm.at[1, pl.ds(base, CPW)], dstbuf)
    plsc.subcore_barrier()

    rows = (rows0, rows1, rows2, rows3)
    sems = (sem0, sem1, sem2, sem3)

    def _start(c, b):
        pltpu.async_copy(y1_hbm.at[srcbuf.at[c]], rows[b], sems[b])

    def _wait(b):
        pltpu.make_async_copy(y1_hbm.at[srcbuf.at[0]], rows[b], sems[b]).wait()

    def _scat(c, b):
        pltpu.sync_copy(rows[b], acc_sh.at[dstbuf.at[c]], add=True)

    for b in range(_NB):
        _start(b, b)

    def _lp(j, c):
        c0 = j * _NB
        for b in range(_NB):
            _wait(b)
            _scat(c0 + b, b)
            _start(c0 + b + _NB, b)
        return c

    lax.fori_loop(0, CPW // _NB - 1, _lp, 0)
    for b in range(_NB):
        _wait(b)
        _scat(CPW - _NB + b, b)

    plsc.subcore_barrier()
    stripe = pl.ds(sid * (NPAD // NS), NPAD // NS)

    @pl.when(cid == 0)
    def _():
        pltpu.sync_copy(acc_sh.at[stripe], out0_hbm.at[stripe])

    @pl.when(cid == 1)
    def _():
        pltpu.sync_copy(acc_sh.at[stripe], out1_hbm.at[stripe])


def _sc_acc2_body(y2_hbm, ei_hbm, out0_hbm, out1_hbm,
                  srcv, dstv, y2buf, accbuf, stripebuf, outbuf, partials_sh):
    cid = lax.axis_index("c")
    sid = lax.axis_index("s")
    wid = cid * NS + sid

    def _zero(j, c):
        accbuf[pl.ds(j * 16, 16)] = jnp.zeros((16,), jnp.float32)
        return c

    lax.fori_loop(0, NPAD // 16, _zero, 0, unroll=8)

    pltpu.sync_copy(y2_hbm, y2buf)
    pltpu.sync_copy(ei_hbm.at[pl.ds(wid * EPW, EPW)], srcv)
    pltpu.sync_copy(ei_hbm.at[pl.ds(E + wid * EPW, EPW)], dstv)

    def _step(j, c):
        sv = srcv[pl.ds(j * 16, 16)]
        dv = dstv[pl.ds(j * 16, 16)]
        vals = plsc.load_gather(y2buf, [sv])
        plsc.addupdate_scatter(accbuf, [dv], vals)
        return c

    lax.fori_loop(0, EPW // 16, _step, 0, unroll=8)
    _reduce_partials(sid, cid, accbuf, partials_sh, stripebuf, outbuf,
                     out0_hbm, out1_hbm)


@functools.cache
def _get_sc_kernels():
    mesh = plsc.VectorSubcoreMesh(core_axis_name="c", subcore_axis_name="s")
    cparams = pltpu.CompilerParams(needs_layout_passes=False)
    sc_deg = pl.kernel(
        _sc_deg_body,
        out_type=[
            jax.ShapeDtypeStruct((NPAD,), jnp.float32),
            jax.ShapeDtypeStruct((NPAD,), jnp.float32),
        ],
        mesh=mesh,
        compiler_params=cparams,
        scratch_types=[
            pltpu.VMEM((EPW,), jnp.int32),
            pltpu.VMEM((NPAD,), jnp.float32),
            pltpu.VMEM((NS, NPAD // NS), jnp.float32),
            pltpu.VMEM((NPAD // NS,), jnp.float32),
            pltpu.VMEM_SHARED((NS, NPAD), jnp.float32),
        ],
    )
    cparams_nt = pltpu.CompilerParams(
        needs_layout_passes=False, use_tc_tiling_on_sc=False
    )
    sc_scatter = pl.kernel(
        _sc_scatter_body,
        out_type=[
            jax.ShapeDtypeStruct((NPAD, H), jnp.bfloat16),
            jax.ShapeDtypeStruct((NPAD, H), jnp.bfloat16),
        ],
        mesh=mesh,
        compiler_params=cparams_nt,
        scratch_types=[
            pltpu.VMEM((CPW, CHUNK), jnp.int32),
            pltpu.VMEM((CPW, CHUNK), jnp.int32),
            pltpu.VMEM((CHUNK, H), jnp.bfloat16),
            pltpu.VMEM((CHUNK, H), jnp.bfloat16),
            pltpu.VMEM((CHUNK, H), jnp.bfloat16),
            pltpu.VMEM((CHUNK, H), jnp.bfloat16),
            pltpu.VMEM((40, H), jnp.bfloat16),
            pltpu.VMEM_SHARED((NPAD, H), jnp.bfloat16),
            pltpu.SemaphoreType.DMA,
            pltpu.SemaphoreType.DMA,
            pltpu.SemaphoreType.DMA,
            pltpu.SemaphoreType.DMA,
        ],
    )
    sc_acc2 = pl.kernel(
        _sc_acc2_body,
        out_type=[
            jax.ShapeDtypeStruct((NPAD,), jnp.float32),
            jax.ShapeDtypeStruct((NPAD,), jnp.float32),
        ],
        mesh=mesh,
        compiler_params=cparams,
        scratch_types=[
            pltpu.VMEM((EPW,), jnp.int32),
            pltpu.VMEM((EPW,), jnp.int32),
            pltpu.VMEM((NPAD,), jnp.float32),
            pltpu.VMEM((NPAD,), jnp.float32),
            pltpu.VMEM((NS, NPAD // NS), jnp.float32),
            pltpu.VMEM((NPAD // NS,), jnp.float32),
            pltpu.VMEM_SHARED((NS, NPAD), jnp.float32),
        ],
    )
    return sc_deg, sc_scatter, sc_acc2


_BR = 1024
_SR = _BR // 128


def _tc_y1_body(x_ref, w1t_ref, d0_ref, d1_ref, y1b_ref):
    xw = jnp.dot(x_ref[...], w1t_ref[...], preferred_element_type=jnp.float32)
    deg = d0_ref[...] + d1_ref[...] + 1.0
    dinv3 = lax.rsqrt(deg)[:, :, None]
    y1 = (dinv3 * jnp.reshape(xw, (_SR, 128, H))).reshape(_BR, H)
    y1b_ref[...] = y1.astype(jnp.bfloat16)


def _tc_y2_body(p0_ref, p1_ref, y1b_ref, d0_ref, d1_ref, w2_ref, y2_ref):
    deg = d0_ref[...] + d1_ref[...] + 1.0
    dinv3 = lax.rsqrt(deg)[:, :, None]
    acc = (p0_ref[...].astype(jnp.float32) + p1_ref[...].astype(jnp.float32)
           + y1b_ref[...].astype(jnp.float32))
    pre3 = jnp.reshape(acc, (_SR, 128, H))
    h3 = jnp.tanh(dinv3 * pre3)
    t = jnp.sum(h3 * w2_ref[...][None], axis=2)
    y2_ref[...] = lax.rsqrt(deg) * t


def _tc_loss_body(a0_ref, a1_ref, y2_ref, d0_ref, d1_ref, batch_ref,
                  label_ref, out_ref):
    dinv = lax.rsqrt(d0_ref[...] + d1_ref[...] + 1.0)
    s = dinv * (a0_ref[...] + a1_ref[...] + y2_ref[...]) * (1.0 / H)
    batch = batch_ref[...]
    total = 0.0
    for g in range(G):
        m = batch == g
        sums = jnp.sum(jnp.where(m, s, 0.0))
        cnt = jnp.sum(jnp.where(m, 1.0, 0.0))
        pooled = sums / jnp.maximum(cnt, 1.0)
        total = total + jnp.log(1.0 + jnp.exp(-pooled * label_ref[0, g]))
    out_ref[...] = jnp.full((1, 1), 1.0 / G, jnp.float32) * total


def kernel(x, edge_index, batch, label, W1, W2):
    sc_deg, sc_scatter, sc_acc2 = _get_sc_kernels()

    ei3 = edge_index.reshape(2, NCHUNKS, CHUNK)
    eiflat = edge_index.reshape(2 * E)
    nrows = NPAD // 128

    d0, d1 = sc_deg(eiflat)
    d0f = d0.reshape(nrows, 128)
    d1f = d1.reshape(nrows, 128)

    y1b = pl.pallas_call(
        _tc_y1_body,
        grid=(NPAD // _BR,),
        in_specs=[
            pl.BlockSpec((_BR, D), lambda j: (j, 0)),
            pl.BlockSpec((D, H), lambda j: (0, 0)),
            pl.BlockSpec((_SR, 128), lambda j: (j, 0)),
            pl.BlockSpec((_SR, 128), lambda j: (j, 0)),
        ],
        out_specs=pl.BlockSpec((_BR, H), lambda j: (j, 0)),
        out_shape=jax.ShapeDtypeStruct((NPAD, H), jnp.bfloat16),
    )(x, W1.T, d0f, d1f)

    p0, p1 = sc_scatter(y1b, ei3)

    y2f = pl.pallas_call(
        _tc_y2_body,
        grid=(NPAD // _BR,),
        in_specs=[
            pl.BlockSpec((_BR, H), lambda j: (j, 0)),
            pl.BlockSpec((_BR, H), lambda j: (j, 0)),
            pl.BlockSpec((_BR, H), lambda j: (j, 0)),
            pl.BlockSpec((_SR, 128), lambda j: (j, 0)),
            pl.BlockSpec((_SR, 128), lambda j: (j, 0)),
            pl.BlockSpec((1, H), lambda j: (0, 0)),
        ],
        out_specs=pl.BlockSpec((_SR, 128), lambda j: (j, 0)),
        out_shape=jax.ShapeDtypeStruct((nrows, 128), jnp.float32),
    )(p0, p1, y1b, d0f, d1f, W2)

    a0, a1 = sc_acc2(y2f.reshape(NPAD), eiflat)

    batch_pad = jnp.pad(batch, (0, NPAD - N), constant_values=G)
    loss = pl.pallas_call(
        _tc_loss_body,
        grid=(1,),
        in_specs=[
            pl.BlockSpec((nrows, 128), lambda j: (0, 0)),
            pl.BlockSpec((nrows, 128), lambda j: (0, 0)),
            pl.BlockSpec((nrows, 128), lambda j: (0, 0)),
            pl.BlockSpec((nrows, 128), lambda j: (0, 0)),
            pl.BlockSpec((nrows, 128), lambda j: (0, 0)),
            pl.BlockSpec((nrows, 128), lambda j: (0, 0)),
            pl.BlockSpec((1, G), lambda j: (0, 0)),
        ],
        out_specs=pl.BlockSpec((1, 1), lambda j: (0, 0)),
        out_shape=jax.ShapeDtypeStruct((1, 1), jnp.float32),
    )(a0.reshape(nrows, 128), a1.reshape(nrows, 128), y2f,
      d0f, d1f, batch_pad.reshape(nrows, 128), label.reshape(1, G))

    return loss.reshape(())
